# baseline (speedup 1.0000x reference)
import functools
import math

import jax
import jax.numpy as jnp
from jax.experimental import pallas as pl
from jax.experimental.pallas import tpu as pltpu


def _row_sum(x):
    n = x.shape[-1]
    if n > 128:
        acc = x[:, :128]
        for i in range(1, n // 128):
            acc = acc + x[:, i * 128:(i + 1) * 128]
    else:
        acc = x
    m = acc.shape[-1]
    while m > 1:
        acc = acc[:, :m // 2] + acc[:, m // 2:m]
        m //= 2
    return acc


def _qkv_kernel(cos_ref, sin_ref, h_ref, r_ref, lnw_ref, qnw_ref, knw_ref,
                w_ref, res_ref, qkv_ref, *, eps, hd, nq_blocks, bn):
    j = pl.program_id(1)
    res = h_ref[...] + r_ref[...]
    res_ref[...] = res
    var = _row_sum(res * res) * (1.0 / res.shape[-1])
    hs = res * jax.lax.rsqrt(var + eps) * lnw_ref[...]
    y = jax.lax.dot_general(hs, w_ref[...], (((1,), (0,)), ((), ())),
                            preferred_element_type=jnp.float32)

    half = hd // 2
    cos = cos_ref[...]
    sin = sin_ref[...]

    nw = jnp.where(j == nq_blocks, knw_ref[...], qnw_ref[...])
    heads = []
    n_heads = bn // hd
    for hh in range(n_heads):
        x = y[:, hh * hd:(hh + 1) * hd]
        v2 = jnp.mean(x * x, axis=-1, keepdims=True)
        xn = x * jax.lax.rsqrt(v2 + eps) * nw
        x1 = xn[:, :half]
        x2 = xn[:, half:]
        roped = jnp.concatenate([x1 * cos - x2 * sin, x2 * cos + x1 * sin],
                                axis=-1)
        heads.append(roped)
    yr = jnp.concatenate(heads, axis=-1)
    qkv_ref[...] = jnp.where(j <= nq_blocks, yr, y)


def _attn_kernel(q_ref, k_ref, v_ref, o_ref, *, scale, bq, t):
    i = pl.program_id(1)
    q = q_ref[...]
    k = k_ref[...]
    v = v_ref[...]
    s = jax.lax.dot_general(q, k, (((1,), (1,)), ((), ())),
                            preferred_element_type=jnp.float32) * scale
    row = i * bq + jax.lax.broadcasted_iota(jnp.int32, (bq, t), 0)
    col = jax.lax.broadcasted_iota(jnp.int32, (bq, t), 1)
    s = jnp.where(row >= col, s, jnp.float32(-1e30))
    m = jnp.max(s, axis=-1, keepdims=True)
    p = jnp.exp(s - m)
    l = _row_sum(p)
    pn = p / l
    o_ref[...] = jax.lax.dot_general(pn, v, (((1,), (0,)), ((), ())),
                                     preferred_element_type=jnp.float32)


def _proj_kernel(o_ref, w_ref, res_ref, out_ref):
    out_ref[...] = jax.lax.dot_general(
        o_ref[...], w_ref[...], (((1,), (0,)), ((), ())),
        preferred_element_type=jnp.float32) + res_ref[...]


def _router_kernel(x_ref, lnw_ref, gw_ref, hs_ref, cw_ref, *, eps, ne):
    x = x_ref[...]
    var = _row_sum(x * x) * (1.0 / x.shape[-1])
    hs = x * jax.lax.rsqrt(var + eps) * lnw_ref[...]
    hs_ref[...] = hs
    logits = jax.lax.dot_general(hs, gw_ref[...], (((1,), (0,)), ((), ())),
                                 preferred_element_type=jnp.float32)
    mx = jnp.max(logits, axis=-1, keepdims=True)
    ex = jnp.exp(logits - mx)
    probs = ex / jnp.sum(ex, axis=-1, keepdims=True)
    col = jax.lax.broadcasted_iota(jnp.int32, probs.shape, 1)
    m1 = jnp.max(probs, axis=-1, keepdims=True)
    idx1 = jnp.min(jnp.where(probs == m1, col, ne), axis=-1, keepdims=True)
    oh1 = (col == idx1).astype(jnp.float32)
    p2 = jnp.where(col == idx1, -jnp.float32(1.0), probs)
    m2 = jnp.max(p2, axis=-1, keepdims=True)
    idx2 = jnp.min(jnp.where(p2 == m2, col, ne), axis=-1, keepdims=True)
    oh2 = (col == idx2).astype(jnp.float32)
    cw = (m1 * oh1 + m2 * oh2) / (m1 + m2)
    cw_ref[...] = cw


def _moe_kernel(x_ref, cw_ref, wg_ref, wu_ref, wd_ref, out_ref):
    e = pl.program_id(1)
    d = pl.program_id(2)

    @pl.when((e == 0) & (d == 0))
    def _():
        out_ref[...] = jnp.zeros_like(out_ref)

    x = x_ref[...]
    g = jax.lax.dot_general(x, wg_ref[0], (((1,), (0,)), ((), ())),
                            preferred_element_type=jnp.float32)
    u = jax.lax.dot_general(x, wu_ref[0], (((1,), (0,)), ((), ())),
                            preferred_element_type=jnp.float32)
    act = (g * jax.lax.logistic(g)) * u
    cw = cw_ref[...]
    col = jax.lax.broadcasted_iota(jnp.int32, cw.shape, 1)
    ce = jnp.sum(jnp.where(col == e, cw, 0.0), axis=-1, keepdims=True)
    y = jax.lax.dot_general(act, wd_ref[0], (((1,), (0,)), ((), ())),
                            preferred_element_type=jnp.float32)
    out_ref[...] += ce * y


def _pipeline(positions, hidden_states, residual, input_ln_w, post_ln_w,
              q_norm_w, k_norm_w, Wqkv, Wo, gate_w, Wg, Wu, Wd):
    T, H = hidden_states.shape
    E, _, DFF = Wg.shape
    NQKV = Wqkv.shape[1]
    NO = Wo.shape[0]
    HD = q_norm_w.shape[0]
    NH = NO // HD
    NKV = (NQKV - NO) // (2 * HD)
    EPS = 1e-6
    THETA = 1000000.0

    BT = min(256, T)
    BN = NKV * HD
    nq_blocks = NH * HD // BN
    n_col_blocks = NQKV // BN

    half = HD // 2
    inv_freq = 1.0 / (THETA ** (jnp.arange(0, HD, 2, dtype=jnp.float32) / HD))
    ang = positions.astype(jnp.float32)[:, None] * inv_freq[None, :]
    cos_t = jnp.cos(ang)
    sin_t = jnp.sin(ang)
    lnw2 = input_ln_w.reshape(1, H)
    qnw2 = q_norm_w.reshape(1, HD)
    knw2 = k_norm_w.reshape(1, HD)

    res, qkv = pl.pallas_call(
        functools.partial(_qkv_kernel, eps=EPS, hd=HD,
                          nq_blocks=nq_blocks, bn=BN),
        grid=(T // BT, n_col_blocks),
        in_specs=[
            pl.BlockSpec((BT, half), lambda i, j: (i, 0)),
            pl.BlockSpec((BT, half), lambda i, j: (i, 0)),
            pl.BlockSpec((BT, H), lambda i, j: (i, 0)),
            pl.BlockSpec((BT, H), lambda i, j: (i, 0)),
            pl.BlockSpec((1, H), lambda i, j: (0, 0)),
            pl.BlockSpec((1, HD), lambda i, j: (0, 0)),
            pl.BlockSpec((1, HD), lambda i, j: (0, 0)),
            pl.BlockSpec((H, BN), lambda i, j: (0, j)),
        ],
        out_specs=[
            pl.BlockSpec((BT, H), lambda i, j: (i, 0)),
            pl.BlockSpec((BT, BN), lambda i, j: (i, j)),
        ],
        out_shape=[
            jax.ShapeDtypeStruct((T, H), jnp.float32),
            jax.ShapeDtypeStruct((T, NQKV), jnp.float32),
        ],
    )(cos_t, sin_t, hidden_states, residual, lnw2, qnw2, knw2, Wqkv)

    BQ = min(256, T)
    grp = NH // NKV
    scale = HD ** -0.5
    k_base = NH
    v_base = NH + NKV

    attn = pl.pallas_call(
        functools.partial(_attn_kernel, scale=scale, bq=BQ, t=T),
        grid=(NH, T // BQ),
        in_specs=[
            pl.BlockSpec((BQ, HD), lambda h, i: (i, h)),
            pl.BlockSpec((T, HD), lambda h, i: (0, k_base + h // grp)),
            pl.BlockSpec((T, HD), lambda h, i: (0, v_base + h // grp)),
        ],
        out_specs=pl.BlockSpec((BQ, HD), lambda h, i: (i, h)),
        out_shape=jax.ShapeDtypeStruct((T, NH * HD), jnp.float32),
    )(qkv, qkv, qkv)

    BJ = min(512, H)
    res2 = pl.pallas_call(
        _proj_kernel,
        grid=(T // BT, H // BJ),
        in_specs=[
            pl.BlockSpec((BT, NO), lambda i, j: (i, 0)),
            pl.BlockSpec((NO, BJ), lambda i, j: (0, j)),
            pl.BlockSpec((BT, BJ), lambda i, j: (i, j)),
        ],
        out_specs=pl.BlockSpec((BT, BJ), lambda i, j: (i, j)),
        out_shape=jax.ShapeDtypeStruct((T, H), jnp.float32),
    )(attn, Wo, res)

    hs2, cw = pl.pallas_call(
        functools.partial(_router_kernel, eps=EPS, ne=E),
        grid=(T // BT,),
        in_specs=[
            pl.BlockSpec((BT, H), lambda i: (i, 0)),
            pl.BlockSpec((1, H), lambda i: (0, 0)),
            pl.BlockSpec((H, E), lambda i: (0, 0)),
        ],
        out_specs=[
            pl.BlockSpec((BT, H), lambda i: (i, 0)),
            pl.BlockSpec((BT, E), lambda i: (i, 0)),
        ],
        out_shape=[
            jax.ShapeDtypeStruct((T, H), jnp.float32),
            jax.ShapeDtypeStruct((T, E), jnp.float32),
        ],
    )(res2, post_ln_w.reshape(1, H), gate_w)

    BD = min(256, DFF)
    out = pl.pallas_call(
        _moe_kernel,
        grid=(T // BT, E, DFF // BD),
        in_specs=[
            pl.BlockSpec((BT, H), lambda i, e, d: (i, 0)),
            pl.BlockSpec((BT, E), lambda i, e, d: (i, 0)),
            pl.BlockSpec((1, H, BD), lambda i, e, d: (e, 0, d)),
            pl.BlockSpec((1, H, BD), lambda i, e, d: (e, 0, d)),
            pl.BlockSpec((1, BD, H), lambda i, e, d: (e, d, 0)),
        ],
        out_specs=pl.BlockSpec((BT, H), lambda i, e, d: (i, 0)),
        out_shape=jax.ShapeDtypeStruct((T, H), jnp.float32),
    )(hs2, cw, Wg, Wu, Wd)

    return out, res2


def kernel(positions, hidden_states, residual, input_ln_w, post_ln_w,
           q_norm_w, k_norm_w, Wqkv, Wo, gate_w, Wg, Wu, Wd):
    return _pipeline(
        positions, hidden_states, residual, input_ln_w, post_ln_w,
        q_norm_w, k_norm_w, Wqkv, Wo, gate_w, Wg, Wu, Wd)


# speedup vs baseline: 1.1898x; 1.1898x over previous
import functools
import math

import jax
import jax.numpy as jnp
from jax.experimental import pallas as pl
from jax.experimental.pallas import tpu as pltpu


def _row_sum(x):
    n = x.shape[-1]
    if n > 128:
        acc = x[:, :128]
        for i in range(1, n // 128):
            acc = acc + x[:, i * 128:(i + 1) * 128]
    else:
        acc = x
    m = acc.shape[-1]
    while m > 1:
        acc = acc[:, :m // 2] + acc[:, m // 2:m]
        m //= 2
    return acc


def _qkv_kernel(cos_ref, sin_ref, h_ref, r_ref, lnw_ref, qnw_ref, knw_ref,
                w_ref, res_ref, qkv_ref, *, eps, hd, nq_blocks, bn):
    j = pl.program_id(1)
    res = h_ref[...] + r_ref[...]
    res_ref[...] = res
    var = _row_sum(res * res) * (1.0 / res.shape[-1])
    hs = res * jax.lax.rsqrt(var + eps) * lnw_ref[...]
    y = jax.lax.dot_general(hs, w_ref[...], (((1,), (0,)), ((), ())),
                            preferred_element_type=jnp.float32)

    half = hd // 2
    cos = cos_ref[...]
    sin = sin_ref[...]

    nw = jnp.where(j == nq_blocks, knw_ref[...], qnw_ref[...])
    heads = []
    n_heads = bn // hd
    for hh in range(n_heads):
        x = y[:, hh * hd:(hh + 1) * hd]
        v2 = jnp.mean(x * x, axis=-1, keepdims=True)
        xn = x * jax.lax.rsqrt(v2 + eps) * nw
        x1 = xn[:, :half]
        x2 = xn[:, half:]
        roped = jnp.concatenate([x1 * cos - x2 * sin, x2 * cos + x1 * sin],
                                axis=-1)
        heads.append(roped)
    yr = jnp.concatenate(heads, axis=-1)
    qkv_ref[...] = jnp.where(j <= nq_blocks, yr, y)


def _attn_kernel(q_ref, k_ref, v_ref, o_ref, *, scale, bq, t):
    i = pl.program_id(1)
    q = q_ref[...]
    k = k_ref[...]
    v = v_ref[...]
    s = jax.lax.dot_general(q, k, (((1,), (1,)), ((), ())),
                            preferred_element_type=jnp.float32) * scale
    row = i * bq + jax.lax.broadcasted_iota(jnp.int32, (bq, t), 0)
    col = jax.lax.broadcasted_iota(jnp.int32, (bq, t), 1)
    s = jnp.where(row >= col, s, jnp.float32(-1e30))
    m = jnp.max(s, axis=-1, keepdims=True)
    p = jnp.exp(s - m)
    l = _row_sum(p)
    pn = p / l
    o_ref[...] = jax.lax.dot_general(pn, v, (((1,), (0,)), ((), ())),
                                     preferred_element_type=jnp.float32)


def _proj_kernel(o_ref, w_ref, res_ref, out_ref):
    out_ref[...] = jax.lax.dot_general(
        o_ref[...], w_ref[...], (((1,), (0,)), ((), ())),
        preferred_element_type=jnp.float32) + res_ref[...]


def _router_kernel(x_ref, lnw_ref, gw_ref, hs_ref, cw_ref, *, eps, ne):
    x = x_ref[...]
    var = _row_sum(x * x) * (1.0 / x.shape[-1])
    hs = x * jax.lax.rsqrt(var + eps) * lnw_ref[...]
    hs_ref[...] = hs
    logits = jax.lax.dot_general(hs, gw_ref[...], (((1,), (0,)), ((), ())),
                                 preferred_element_type=jnp.float32)
    mx = jnp.max(logits, axis=-1, keepdims=True)
    ex = jnp.exp(logits - mx)
    probs = ex / jnp.sum(ex, axis=-1, keepdims=True)
    col = jax.lax.broadcasted_iota(jnp.int32, probs.shape, 1)
    m1 = jnp.max(probs, axis=-1, keepdims=True)
    idx1 = jnp.min(jnp.where(probs == m1, col, ne), axis=-1, keepdims=True)
    oh1 = (col == idx1).astype(jnp.float32)
    p2 = jnp.where(col == idx1, -jnp.float32(1.0), probs)
    m2 = jnp.max(p2, axis=-1, keepdims=True)
    idx2 = jnp.min(jnp.where(p2 == m2, col, ne), axis=-1, keepdims=True)
    oh2 = (col == idx2).astype(jnp.float32)
    cw = (m1 * oh1 + m2 * oh2) / (m1 + m2)
    cw_ref[...] = cw


def _moe_kernel(x_ref, cw_ref, wg_ref, wu_ref, wd_ref, out_ref, *, bt):
    e = pl.program_id(0)
    d = pl.program_id(1)

    @pl.when((e == 0) & (d == 0))
    def _():
        out_ref[...] = jnp.zeros_like(out_ref)

    wg = wg_ref[0]
    wu = wu_ref[0]
    wd = wd_ref[0]
    cw_all = cw_ref[...]
    col = jax.lax.broadcasted_iota(jnp.int32, cw_all.shape, 1)
    ce_all = jnp.sum(jnp.where(col == e, cw_all, 0.0), axis=-1, keepdims=True)
    nrows = x_ref.shape[0] // bt
    for i in range(nrows):
        x = x_ref[pl.ds(i * bt, bt), :]
        g = jax.lax.dot_general(x, wg, (((1,), (0,)), ((), ())),
                                preferred_element_type=jnp.float32)
        u = jax.lax.dot_general(x, wu, (((1,), (0,)), ((), ())),
                                preferred_element_type=jnp.float32)
        act = (g * jax.lax.logistic(g)) * u
        y = jax.lax.dot_general(act, wd, (((1,), (0,)), ((), ())),
                                preferred_element_type=jnp.float32)
        out_ref[pl.ds(i * bt, bt), :] += ce_all[i * bt:(i + 1) * bt, :] * y


def _pipeline(positions, hidden_states, residual, input_ln_w, post_ln_w,
              q_norm_w, k_norm_w, Wqkv, Wo, gate_w, Wg, Wu, Wd):
    T, H = hidden_states.shape
    E, _, DFF = Wg.shape
    NQKV = Wqkv.shape[1]
    NO = Wo.shape[0]
    HD = q_norm_w.shape[0]
    NH = NO // HD
    NKV = (NQKV - NO) // (2 * HD)
    EPS = 1e-6
    THETA = 1000000.0

    BT = min(256, T)
    BN = NKV * HD
    nq_blocks = NH * HD // BN
    n_col_blocks = NQKV // BN

    half = HD // 2
    inv_freq = 1.0 / (THETA ** (jnp.arange(0, HD, 2, dtype=jnp.float32) / HD))
    ang = positions.astype(jnp.float32)[:, None] * inv_freq[None, :]
    cos_t = jnp.cos(ang)
    sin_t = jnp.sin(ang)
    lnw2 = input_ln_w.reshape(1, H)
    qnw2 = q_norm_w.reshape(1, HD)
    knw2 = k_norm_w.reshape(1, HD)

    res, qkv = pl.pallas_call(
        functools.partial(_qkv_kernel, eps=EPS, hd=HD,
                          nq_blocks=nq_blocks, bn=BN),
        grid=(T // BT, n_col_blocks),
        in_specs=[
            pl.BlockSpec((BT, half), lambda i, j: (i, 0)),
            pl.BlockSpec((BT, half), lambda i, j: (i, 0)),
            pl.BlockSpec((BT, H), lambda i, j: (i, 0)),
            pl.BlockSpec((BT, H), lambda i, j: (i, 0)),
            pl.BlockSpec((1, H), lambda i, j: (0, 0)),
            pl.BlockSpec((1, HD), lambda i, j: (0, 0)),
            pl.BlockSpec((1, HD), lambda i, j: (0, 0)),
            pl.BlockSpec((H, BN), lambda i, j: (0, j)),
        ],
        out_specs=[
            pl.BlockSpec((BT, H), lambda i, j: (i, 0)),
            pl.BlockSpec((BT, BN), lambda i, j: (i, j)),
        ],
        out_shape=[
            jax.ShapeDtypeStruct((T, H), jnp.float32),
            jax.ShapeDtypeStruct((T, NQKV), jnp.float32),
        ],
    )(cos_t, sin_t, hidden_states, residual, lnw2, qnw2, knw2, Wqkv)

    BQ = min(256, T)
    grp = NH // NKV
    scale = HD ** -0.5
    k_base = NH
    v_base = NH + NKV

    attn = pl.pallas_call(
        functools.partial(_attn_kernel, scale=scale, bq=BQ, t=T),
        grid=(NH, T // BQ),
        in_specs=[
            pl.BlockSpec((BQ, HD), lambda h, i: (i, h)),
            pl.BlockSpec((T, HD), lambda h, i: (0, k_base + h // grp)),
            pl.BlockSpec((T, HD), lambda h, i: (0, v_base + h // grp)),
        ],
        out_specs=pl.BlockSpec((BQ, HD), lambda h, i: (i, h)),
        out_shape=jax.ShapeDtypeStruct((T, NH * HD), jnp.float32),
    )(qkv, qkv, qkv)

    BJ = min(512, H)
    res2 = pl.pallas_call(
        _proj_kernel,
        grid=(T // BT, H // BJ),
        in_specs=[
            pl.BlockSpec((BT, NO), lambda i, j: (i, 0)),
            pl.BlockSpec((NO, BJ), lambda i, j: (0, j)),
            pl.BlockSpec((BT, BJ), lambda i, j: (i, j)),
        ],
        out_specs=pl.BlockSpec((BT, BJ), lambda i, j: (i, j)),
        out_shape=jax.ShapeDtypeStruct((T, H), jnp.float32),
    )(attn, Wo, res)

    hs2, cw = pl.pallas_call(
        functools.partial(_router_kernel, eps=EPS, ne=E),
        grid=(T // BT,),
        in_specs=[
            pl.BlockSpec((BT, H), lambda i: (i, 0)),
            pl.BlockSpec((1, H), lambda i: (0, 0)),
            pl.BlockSpec((H, E), lambda i: (0, 0)),
        ],
        out_specs=[
            pl.BlockSpec((BT, H), lambda i: (i, 0)),
            pl.BlockSpec((BT, E), lambda i: (i, 0)),
        ],
        out_shape=[
            jax.ShapeDtypeStruct((T, H), jnp.float32),
            jax.ShapeDtypeStruct((T, E), jnp.float32),
        ],
    )(res2, post_ln_w.reshape(1, H), gate_w)

    BD = min(256, DFF)
    out = pl.pallas_call(
        functools.partial(_moe_kernel, bt=BT),
        grid=(E, DFF // BD),
        in_specs=[
            pl.BlockSpec((T, H), lambda e, d: (0, 0)),
            pl.BlockSpec((T, E), lambda e, d: (0, 0)),
            pl.BlockSpec((1, H, BD), lambda e, d: (e, 0, d)),
            pl.BlockSpec((1, H, BD), lambda e, d: (e, 0, d)),
            pl.BlockSpec((1, BD, H), lambda e, d: (e, d, 0)),
        ],
        out_specs=pl.BlockSpec((T, H), lambda e, d: (0, 0)),
        out_shape=jax.ShapeDtypeStruct((T, H), jnp.float32),
    )(hs2, cw, Wg, Wu, Wd)

    return out, res2


def kernel(positions, hidden_states, residual, input_ln_w, post_ln_w,
           q_norm_w, k_norm_w, Wqkv, Wo, gate_w, Wg, Wu, Wd):
    return _pipeline(
        positions, hidden_states, residual, input_ln_w, post_ln_w,
        q_norm_w, k_norm_w, Wqkv, Wo, gate_w, Wg, Wu, Wd)


# speedup vs baseline: 1.2577x; 1.0570x over previous
import functools
import math

import jax
import jax.numpy as jnp
from jax.experimental import pallas as pl
from jax.experimental.pallas import tpu as pltpu


def _row_sum(x):
    n = x.shape[-1]
    if n > 128:
        acc = x[:, :128]
        for i in range(1, n // 128):
            acc = acc + x[:, i * 128:(i + 1) * 128]
    else:
        acc = x
    m = acc.shape[-1]
    while m > 1:
        acc = acc[:, :m // 2] + acc[:, m // 2:m]
        m //= 2
    return acc


def _addnorm_kernel(h_ref, r_ref, lnw_ref, res_ref, hs_ref, *, eps):
    res = h_ref[...] + r_ref[...]
    res_ref[...] = res
    var = _row_sum(res * res) * (1.0 / res.shape[-1])
    hs_ref[...] = res * jax.lax.rsqrt(var + eps) * lnw_ref[...]


def _qkv_kernel(cos_ref, sin_ref, hs_ref, qnw_ref, knw_ref,
                w_ref, qkv_ref, *, eps, hd, nq_blocks, bn, bt):
    j = pl.program_id(0)
    i = pl.program_id(1)
    hs = hs_ref[pl.ds(i * bt, bt), :]
    y = jax.lax.dot_general(hs, w_ref[...], (((1,), (0,)), ((), ())),
                            preferred_element_type=jnp.float32)

    half = hd // 2
    cos = cos_ref[...]
    sin = sin_ref[...]

    eps = jnp.float32(eps)
    nw = jnp.where(j == nq_blocks, knw_ref[...], qnw_ref[...])
    heads = []
    n_heads = bn // hd
    for hh in range(n_heads):
        x = y[:, hh * hd:(hh + 1) * hd]
        v2 = jnp.mean(x * x, axis=-1, keepdims=True)
        xn = x * jax.lax.rsqrt(v2 + eps) * nw
        x1 = xn[:, :half]
        x2 = xn[:, half:]
        roped = jnp.concatenate([x1 * cos - x2 * sin, x2 * cos + x1 * sin],
                                axis=-1)
        heads.append(roped)
    yr = jnp.concatenate(heads, axis=-1)
    qkv_ref[...] = jnp.where(j <= nq_blocks, yr, y)


def _attn_kernel(q_ref, k_ref, v_ref, o_ref, *, scale, bq, t):
    i = pl.program_id(1)
    q = q_ref[...]
    k = k_ref[...]
    v = v_ref[...]
    s = jax.lax.dot_general(q, k, (((1,), (1,)), ((), ())),
                            preferred_element_type=jnp.float32) * scale
    row = i * bq + jax.lax.broadcasted_iota(jnp.int32, (bq, t), 0)
    col = jax.lax.broadcasted_iota(jnp.int32, (bq, t), 1)
    s = jnp.where(row >= col, s, jnp.float32(-1e30))
    m = jnp.max(s, axis=-1, keepdims=True)
    p = jnp.exp(s - m)
    l = _row_sum(p)
    pn = p / l
    o_ref[...] = jax.lax.dot_general(pn, v, (((1,), (0,)), ((), ())),
                                     preferred_element_type=jnp.float32)


def _proj_kernel(o_ref, w_ref, res_ref, out_ref, *, bt):
    i = pl.program_id(1)
    o = o_ref[pl.ds(i * bt, bt), :]
    out_ref[...] = jax.lax.dot_general(
        o, w_ref[...], (((1,), (0,)), ((), ())),
        preferred_element_type=jnp.float32) + res_ref[...]


def _router_kernel(x_ref, lnw_ref, gw_ref, hs_ref, cw_ref, *, eps, ne):
    x = x_ref[...]
    var = _row_sum(x * x) * (1.0 / x.shape[-1])
    hs = x * jax.lax.rsqrt(var + eps) * lnw_ref[...]
    hs_ref[...] = hs
    logits = jax.lax.dot_general(hs, gw_ref[...], (((1,), (0,)), ((), ())),
                                 preferred_element_type=jnp.float32)
    mx = jnp.max(logits, axis=-1, keepdims=True)
    ex = jnp.exp(logits - mx)
    probs = ex / jnp.sum(ex, axis=-1, keepdims=True)
    col = jax.lax.broadcasted_iota(jnp.int32, probs.shape, 1)
    m1 = jnp.max(probs, axis=-1, keepdims=True)
    idx1 = jnp.min(jnp.where(probs == m1, col, ne), axis=-1, keepdims=True)
    oh1 = (col == idx1).astype(jnp.float32)
    p2 = jnp.where(col == idx1, -jnp.float32(1.0), probs)
    m2 = jnp.max(p2, axis=-1, keepdims=True)
    idx2 = jnp.min(jnp.where(p2 == m2, col, ne), axis=-1, keepdims=True)
    oh2 = (col == idx2).astype(jnp.float32)
    cw = (m1 * oh1 + m2 * oh2) / (m1 + m2)
    cw_ref[...] = cw


def _moe_kernel(x_ref, cw_ref, wg_ref, wu_ref, wd_ref, out_ref, *, bt):
    e = pl.program_id(0)
    d = pl.program_id(1)

    @pl.when((e == 0) & (d == 0))
    def _():
        out_ref[...] = jnp.zeros_like(out_ref)

    wg = wg_ref[0]
    wu = wu_ref[0]
    wd = wd_ref[0]
    cw_all = cw_ref[...]
    col = jax.lax.broadcasted_iota(jnp.int32, cw_all.shape, 1)
    ce_all = jnp.sum(jnp.where(col == e, cw_all, 0.0), axis=-1, keepdims=True)
    nrows = x_ref.shape[0] // bt
    for i in range(nrows):
        x = x_ref[pl.ds(i * bt, bt), :]
        g = jax.lax.dot_general(x, wg, (((1,), (0,)), ((), ())),
                                preferred_element_type=jnp.float32)
        u = jax.lax.dot_general(x, wu, (((1,), (0,)), ((), ())),
                                preferred_element_type=jnp.float32)
        act = (g * jax.lax.logistic(g)) * u
        y = jax.lax.dot_general(act, wd, (((1,), (0,)), ((), ())),
                                preferred_element_type=jnp.float32)
        out_ref[pl.ds(i * bt, bt), :] += ce_all[i * bt:(i + 1) * bt, :] * y


def _pipeline(positions, hidden_states, residual, input_ln_w, post_ln_w,
              q_norm_w, k_norm_w, Wqkv, Wo, gate_w, Wg, Wu, Wd):
    T, H = hidden_states.shape
    E, _, DFF = Wg.shape
    NQKV = Wqkv.shape[1]
    NO = Wo.shape[0]
    HD = q_norm_w.shape[0]
    NH = NO // HD
    NKV = (NQKV - NO) // (2 * HD)
    EPS = 1e-6
    THETA = 1000000.0

    BT = min(256, T)
    BN = NKV * HD
    nq_blocks = NH * HD // BN
    n_col_blocks = NQKV // BN

    half = HD // 2
    inv_freq = 1.0 / (THETA ** (jnp.arange(0, HD, 2, dtype=jnp.float32) / HD))
    ang = positions.astype(jnp.float32)[:, None] * inv_freq[None, :]
    cos_t = jnp.cos(ang)
    sin_t = jnp.sin(ang)
    lnw2 = input_ln_w.reshape(1, H)
    qnw2 = q_norm_w.reshape(1, HD)
    knw2 = k_norm_w.reshape(1, HD)

    res, hs = pl.pallas_call(
        functools.partial(_addnorm_kernel, eps=EPS),
        grid=(T // BT,),
        in_specs=[
            pl.BlockSpec((BT, H), lambda i: (i, 0)),
            pl.BlockSpec((BT, H), lambda i: (i, 0)),
            pl.BlockSpec((1, H), lambda i: (0, 0)),
        ],
        out_specs=[
            pl.BlockSpec((BT, H), lambda i: (i, 0)),
            pl.BlockSpec((BT, H), lambda i: (i, 0)),
        ],
        out_shape=[
            jax.ShapeDtypeStruct((T, H), jnp.float32),
            jax.ShapeDtypeStruct((T, H), jnp.float32),
        ],
    )(hidden_states, residual, lnw2)

    qkv = pl.pallas_call(
        functools.partial(_qkv_kernel, eps=EPS, hd=HD,
                          nq_blocks=nq_blocks, bn=BN, bt=BT),
        grid=(n_col_blocks, T // BT),
        in_specs=[
            pl.BlockSpec((BT, half), lambda j, i: (i, 0)),
            pl.BlockSpec((BT, half), lambda j, i: (i, 0)),
            pl.BlockSpec((T, H), lambda j, i: (0, 0)),
            pl.BlockSpec((1, HD), lambda j, i: (0, 0)),
            pl.BlockSpec((1, HD), lambda j, i: (0, 0)),
            pl.BlockSpec((H, BN), lambda j, i: (0, j)),
        ],
        out_specs=pl.BlockSpec((BT, BN), lambda j, i: (i, j)),
        out_shape=jax.ShapeDtypeStruct((T, NQKV), jnp.float32),
    )(cos_t, sin_t, hs, qnw2, knw2, Wqkv)

    BQ = min(256, T)
    grp = NH // NKV
    scale = HD ** -0.5
    k_base = NH
    v_base = NH + NKV

    attn = pl.pallas_call(
        functools.partial(_attn_kernel, scale=scale, bq=BQ, t=T),
        grid=(NH, T // BQ),
        in_specs=[
            pl.BlockSpec((BQ, HD), lambda h, i: (i, h)),
            pl.BlockSpec((T, HD), lambda h, i: (0, k_base + h // grp)),
            pl.BlockSpec((T, HD), lambda h, i: (0, v_base + h // grp)),
        ],
        out_specs=pl.BlockSpec((BQ, HD), lambda h, i: (i, h)),
        out_shape=jax.ShapeDtypeStruct((T, NH * HD), jnp.float32),
    )(qkv, qkv, qkv)

    BJ = min(256, H)
    res2 = pl.pallas_call(
        functools.partial(_proj_kernel, bt=BT),
        grid=(H // BJ, T // BT),
        in_specs=[
            pl.BlockSpec((T, NO), lambda j, i: (0, 0)),
            pl.BlockSpec((NO, BJ), lambda j, i: (0, j)),
            pl.BlockSpec((BT, BJ), lambda j, i: (i, j)),
        ],
        out_specs=pl.BlockSpec((BT, BJ), lambda j, i: (i, j)),
        out_shape=jax.ShapeDtypeStruct((T, H), jnp.float32),
    )(attn, Wo, res)

    hs2, cw = pl.pallas_call(
        functools.partial(_router_kernel, eps=EPS, ne=E),
        grid=(T // BT,),
        in_specs=[
            pl.BlockSpec((BT, H), lambda i: (i, 0)),
            pl.BlockSpec((1, H), lambda i: (0, 0)),
            pl.BlockSpec((H, E), lambda i: (0, 0)),
        ],
        out_specs=[
            pl.BlockSpec((BT, H), lambda i: (i, 0)),
            pl.BlockSpec((BT, E), lambda i: (i, 0)),
        ],
        out_shape=[
            jax.ShapeDtypeStruct((T, H), jnp.float32),
            jax.ShapeDtypeStruct((T, E), jnp.float32),
        ],
    )(res2, post_ln_w.reshape(1, H), gate_w)

    BD = min(256, DFF)
    out = pl.pallas_call(
        functools.partial(_moe_kernel, bt=BT),
        grid=(E, DFF // BD),
        in_specs=[
            pl.BlockSpec((T, H), lambda e, d: (0, 0)),
            pl.BlockSpec((T, E), lambda e, d: (0, 0)),
            pl.BlockSpec((1, H, BD), lambda e, d: (e, 0, d)),
            pl.BlockSpec((1, H, BD), lambda e, d: (e, 0, d)),
            pl.BlockSpec((1, BD, H), lambda e, d: (e, d, 0)),
        ],
        out_specs=pl.BlockSpec((T, H), lambda e, d: (0, 0)),
        out_shape=jax.ShapeDtypeStruct((T, H), jnp.float32),
    )(hs2, cw, Wg, Wu, Wd)

    return out, res2


def kernel(positions, hidden_states, residual, input_ln_w, post_ln_w,
           q_norm_w, k_norm_w, Wqkv, Wo, gate_w, Wg, Wu, Wd):
    return _pipeline(
        positions, hidden_states, residual, input_ln_w, post_ln_w,
        q_norm_w, k_norm_w, Wqkv, Wo, gate_w, Wg, Wu, Wd)
